# Initial kernel scaffold; baseline (speedup 1.0000x reference)
import jax
import jax.numpy as jnp
from jax.experimental import pallas as pl


def kernel(x, edge_index, Ws1, Wn1, b1, Ws2, Wn2, b2, Ws3, Wn3, b3):
    raise NotImplementedError("write your pallas kernel here")


# speedup vs baseline: 5.6746x; 5.6746x over previous
import functools

import jax
import jax.numpy as jnp
from jax import lax
from jax.experimental import pallas as pl
from jax.experimental.pallas import tpu as pltpu
from jax.experimental.pallas import tpu_sc as plsc

NC = 2
NS = 16
LANES = 16

_K = 80
_ZR = 40
_ZT = 10
_RED_T = 5
_RED_N = 2048


@functools.lru_cache(maxsize=None)
def _make_sc_agg(N, E, W, mode):
    EPC = E if mode == "l2" else E // NC
    EP = EPC // NS
    NCHUNK = EP // _K
    ZROWS = N // _ZT
    NP = _RED_T * _RED_N
    assert EPC % (NS * _K) == 0
    assert ZROWS % _ZR == 0 and ZROWS % 8 == 0 and N % LANES == 0
    assert NP >= N and _RED_N % LANES == 0

    mesh = plsc.VectorSubcoreMesh(core_axis_name="c", subcore_axis_name="s")

    scratch = [
        pltpu.VMEM_SHARED((N, W), jnp.float32),
        pltpu.HBM((2, NS, 1, NP), jnp.float32),
        pltpu.VMEM((_K,), jnp.int32),
        pltpu.VMEM((_K,), jnp.int32),
        pltpu.VMEM((_K, W), jnp.float32),
        pltpu.VMEM((_ZR, W), jnp.float32),
        pltpu.VMEM((NP,), jnp.float32),
        pltpu.VMEM((N,), jnp.float32),
        pltpu.VMEM((_RED_N,), jnp.float32),
        pltpu.VMEM((_RED_N,), jnp.float32),
        pltpu.SemaphoreType.DMA,
    ]

    def body(*refs):
        if mode == "l2":
            (ha, hb, srcarr, dst, invdeg, agg_out, side_out,
             acc, slab, srcv, dstv, rows, zbuf, sidev, invdv,
             red, tmp, sem) = refs
        else:
            (ha, srcarr, dst, agg_out, side_out,
             acc, slab, srcv, dstv, rows, zbuf, sidev, invdv,
             red, tmp, sem) = refs
            hb = ha
            invdeg = None

        c = lax.axis_index("c")
        s = lax.axis_index("s")

        zeros16 = jnp.zeros((LANES,), jnp.float32)

        def zrow(r, _):
            def zcol(j, _):
                zbuf[r, pl.ds(j * LANES, LANES)] = zeros16
                return 0
            return lax.fori_loop(0, W // LANES, zcol, 0)
        lax.fori_loop(0, _ZR, zrow, 0)

        def zside(i, _):
            sidev[pl.ds(i * LANES, LANES)] = zeros16
            return 0
        lax.fori_loop(0, NP // LANES, zside, 0)

        if mode == "l2":
            pltpu.sync_copy(invdeg, invdv)

        @pl.when(s < _ZT)
        def _():
            for b in range(ZROWS // _ZR):
                pltpu.sync_copy(zbuf,
                                acc.at[pl.ds(s * ZROWS + b * _ZR, _ZR)])
        plsc.subcore_barrier()

        ones16 = jnp.ones((LANES,), jnp.float32)
        ebase = s * EP if mode == "l2" else c * EPC + s * EP

        def step(i, _):
            off = ebase + i * _K
            pltpu.sync_copy(srcarr.at[pl.ds(off, _K)], srcv)
            pltpu.sync_copy(dst.at[pl.ds(off, _K)], dstv)
            if mode == "l2":
                @pl.when(c == 0)
                def _():
                    pltpu.async_copy(ha.at[srcv], rows, sem).wait()

                @pl.when(c == 1)
                def _():
                    pltpu.async_copy(hb.at[srcv], rows, sem).wait()
            else:
                pltpu.async_copy(ha.at[srcv], rows, sem).wait()
            pltpu.sync_copy(rows, acc.at[dstv], add=True)

            if mode == "l1":
                for j in range(_K // LANES):
                    d16 = dstv[pl.ds(j * LANES, LANES)]
                    plsc.addupdate_scatter(sidev, [d16], ones16)
            else:
                @pl.when(c == 1)
                def _():
                    for j in range(_K // LANES):
                        d16 = dstv[pl.ds(j * LANES, LANES)]
                        w16 = plsc.load_gather(invdv, [d16])
                        u16 = srcv[pl.ds(j * LANES, LANES)]
                        plsc.addupdate_scatter(sidev, [u16], w16)
            return 0

        lax.fori_loop(0, NCHUNK, step, 0)

        pltpu.sync_copy(sidev, slab.at[c, s, 0])
        plsc.subcore_barrier()

        @pl.when(s < _ZT)
        def _():
            pltpu.sync_copy(acc.at[pl.ds(s * ZROWS, ZROWS)],
                            agg_out.at[c, pl.ds(s * ZROWS, ZROWS)])

        @pl.when(s < _RED_T)
        def _():
            base = s * _RED_N
            pltpu.sync_copy(slab.at[c, 0, 0, pl.ds(base, _RED_N)], red)
            for t in range(1, NS):
                pltpu.sync_copy(slab.at[c, t, 0, pl.ds(base, _RED_N)], tmp)

                def radd(j, _):
                    sl = pl.ds(j * LANES, LANES)
                    red[sl] = red[sl] + tmp[sl]
                    return 0
                lax.fori_loop(0, _RED_N // LANES, radd, 0)
            pltpu.sync_copy(red, side_out.at[c, 0, pl.ds(base, _RED_N)])

    NP = _RED_T * _RED_N
    out_types = (
        jax.ShapeDtypeStruct((2, N, W), jnp.float32),
        jax.ShapeDtypeStruct((2, 1, NP), jnp.float32),
    )
    return pl.kernel(body, out_type=out_types, mesh=mesh,
                     scratch_types=scratch,
                     compiler_params=pltpu.CompilerParams(
                         needs_layout_passes=False))


@functools.lru_cache(maxsize=None)
def _make_tc1(N, D_IN, D_HID, R):
    grid = (N // R,)
    H = D_HID // 2

    def body(x, agg, degt, Ws1, Wn1, b1, h1a_out, h1b_out, invd_out):
        deg = jnp.sum(degt[...], axis=1, keepdims=True)
        inv = 1.0 / jnp.clip(deg, 1.0, None)
        invd_out[...] = inv
        mean = (agg[0] + agg[1]) * inv
        h = (jnp.dot(x[...], Ws1[...], preferred_element_type=jnp.float32)
             + jnp.dot(mean, Wn1[...], preferred_element_type=jnp.float32)
             + b1[...])
        h = jnp.maximum(h, 0.0)
        nrm = jnp.sqrt(jnp.sum(h * h, axis=1, keepdims=True))
        h = h / jnp.maximum(nrm, 1e-12)
        h1a_out[...] = h[:, 0:H]
        h1b_out[...] = h[:, H:2 * H]

    return pl.pallas_call(
        body,
        grid=grid,
        in_specs=[
            pl.BlockSpec((R, D_IN), lambda i: (i, 0)),
            pl.BlockSpec((2, R, D_IN), lambda i: (0, i, 0)),
            pl.BlockSpec((R, 2), lambda i: (i, 0)),
            pl.BlockSpec((D_IN, D_HID), lambda i: (0, 0)),
            pl.BlockSpec((D_IN, D_HID), lambda i: (0, 0)),
            pl.BlockSpec((1, D_HID), lambda i: (0, 0)),
        ],
        out_specs=[
            pl.BlockSpec((R, H), lambda i: (i, 0)),
            pl.BlockSpec((R, H), lambda i: (i, 0)),
            pl.BlockSpec((R, 1), lambda i: (i, 0)),
        ],
        out_shape=[
            jax.ShapeDtypeStruct((N, H), jnp.float32),
            jax.ShapeDtypeStruct((N, H), jnp.float32),
            jax.ShapeDtypeStruct((N, 1), jnp.float32),
        ],
    )


@functools.lru_cache(maxsize=None)
def _make_tc2(N, D_HID, D_OUT, R):
    grid = (N // R,)
    H = D_HID // 2
    nblocks = N // R

    def body(h1a, h1b, agg, cvec, invd, Ws2, Wn2, b2, Ws3, Wn3, b3,
             out, s1, s2):
        i = pl.program_id(0)
        inv = invd[...]
        h = (jnp.dot(h1a[...], Ws2[0:H], preferred_element_type=jnp.float32)
             + jnp.dot(h1b[...], Ws2[H:2 * H],
                       preferred_element_type=jnp.float32)
             + jnp.dot(agg[0] * inv, Wn2[0:H],
                       preferred_element_type=jnp.float32)
             + jnp.dot(agg[1] * inv, Wn2[H:2 * H],
                       preferred_element_type=jnp.float32)
             + b2[...])
        h = jnp.maximum(h, 0.0)
        nrm = jnp.sqrt(jnp.sum(h * h, axis=1, keepdims=True))
        h2 = h / jnp.maximum(nrm, 1e-12)

        @pl.when(i == 0)
        def _():
            s1[...] = jnp.zeros_like(s1)
            s2[...] = jnp.zeros_like(s2)

        s1[...] += jnp.sum(h2, axis=0, keepdims=True)
        s2[...] += jnp.sum(cvec[...] * h2, axis=0, keepdims=True)

        @pl.when(i == nblocks - 1)
        def _():
            out[...] = (jnp.dot(s1[...], Ws3[...],
                                preferred_element_type=jnp.float32)
                        + jnp.dot(s2[...], Wn3[...],
                                  preferred_element_type=jnp.float32)
                        ) * (1.0 / N) + b3[...]

    return pl.pallas_call(
        body,
        grid=grid,
        in_specs=[
            pl.BlockSpec((R, H), lambda i: (i, 0)),
            pl.BlockSpec((R, H), lambda i: (i, 0)),
            pl.BlockSpec((2, R, H), lambda i: (0, i, 0)),
            pl.BlockSpec((R, 1), lambda i: (i, 0)),
            pl.BlockSpec((R, 1), lambda i: (i, 0)),
            pl.BlockSpec((D_HID, D_HID), lambda i: (0, 0)),
            pl.BlockSpec((D_HID, D_HID), lambda i: (0, 0)),
            pl.BlockSpec((1, D_HID), lambda i: (0, 0)),
            pl.BlockSpec((D_HID, D_OUT), lambda i: (0, 0)),
            pl.BlockSpec((D_HID, D_OUT), lambda i: (0, 0)),
            pl.BlockSpec((1, D_OUT), lambda i: (0, 0)),
        ],
        out_specs=pl.BlockSpec((1, D_OUT), lambda i: (0, 0)),
        out_shape=jax.ShapeDtypeStruct((1, D_OUT), jnp.float32),
        scratch_shapes=[
            pltpu.VMEM((1, D_HID), jnp.float32),
            pltpu.VMEM((1, D_HID), jnp.float32),
        ],
    )


def kernel(x, edge_index, Ws1, Wn1, b1, Ws2, Wn2, b2, Ws3, Wn3, b3):
    N, D_IN = x.shape
    E = edge_index.shape[1]
    D_HID = Ws2.shape[0]
    D_OUT = Ws3.shape[1]
    R = 1000

    src = edge_index[0]
    dst = edge_index[1]

    agg1, degp = _make_sc_agg(N, E, D_IN, "l1")(x, src, dst)
    degt = degp[:, 0, :N].T

    h1a, h1b, invd = _make_tc1(N, D_IN, D_HID, R)(
        x, agg1, degt, Ws1, Wn1, b1.reshape(1, D_HID))

    agg2, cside = _make_sc_agg(N, E, D_HID // 2, "l2")(
        h1a, h1b, src, dst, invd.reshape(N))
    cvec = cside[1, 0, :N].reshape(N, 1)

    out = _make_tc2(N, D_HID, D_OUT, R)(
        h1a, h1b, agg2, cvec, invd, Ws2, Wn2, b2.reshape(1, D_HID),
        Ws3, Wn3, b3.reshape(1, D_OUT))
    return out.reshape(D_OUT)


# speedup vs baseline: 8.9814x; 1.5827x over previous
import functools

import jax
import jax.numpy as jnp
from jax import lax
from jax.experimental import pallas as pl
from jax.experimental.pallas import tpu as pltpu
from jax.experimental.pallas import tpu_sc as plsc

NC = 2
NS = 16
LANES = 16

_K = 80
_ZR = 40
_ZT = 10
_RED_T = 5
_RED_N = 2048


@functools.lru_cache(maxsize=None)
def _make_sc_agg(N, E, W, mode):
    EPC = E if mode == "l2" else E // NC
    EP = EPC // NS
    NCHUNK = EP // _K
    ZROWS = N // _ZT
    NP = _RED_T * _RED_N
    assert EPC % (NS * _K) == 0
    assert ZROWS % _ZR == 0 and ZROWS % 8 == 0 and N % LANES == 0
    assert NP >= N and _RED_N % LANES == 0

    mesh = plsc.VectorSubcoreMesh(core_axis_name="c", subcore_axis_name="s")

    scratch = [
        pltpu.VMEM_SHARED((N, W), jnp.float32),
        pltpu.HBM((2, NS, 1, NP), jnp.float32),
        pltpu.VMEM((2, _K), jnp.int32),
        pltpu.VMEM((2, _K), jnp.int32),
        pltpu.VMEM((2, _K, W), jnp.float32),
        pltpu.VMEM((_ZR, W), jnp.float32),
        pltpu.VMEM((NP,), jnp.float32),
        pltpu.VMEM((N,), jnp.float32),
        pltpu.VMEM((_RED_N,), jnp.float32),
        pltpu.VMEM((_RED_N,), jnp.float32),
        pltpu.SemaphoreType.DMA,
        pltpu.SemaphoreType.DMA,
        pltpu.SemaphoreType.DMA,
    ]

    def body(*refs):
        if mode == "l2":
            (ha, hb, srcarr, dst, invdeg, agg_out, side_out,
             acc, slab, srcv, dstv, rows, zbuf, sidev, invdv,
             red, tmp, semg0, semg1, semz) = refs
        else:
            (ha, srcarr, dst, agg_out, side_out,
             acc, slab, srcv, dstv, rows, zbuf, sidev, invdv,
             red, tmp, semg0, semg1, semz) = refs
            hb = ha
            invdeg = None
        semg = (semg0, semg1)

        c = lax.axis_index("c")
        s = lax.axis_index("s")

        zeros16 = jnp.zeros((LANES,), jnp.float32)

        def zrow(r, _):
            def zcol(j, _):
                zbuf[r, pl.ds(j * LANES, LANES)] = zeros16
                return 0
            return lax.fori_loop(0, W // LANES, zcol, 0)
        lax.fori_loop(0, _ZR, zrow, 0)

        def zside(i, _):
            sidev[pl.ds(i * LANES, LANES)] = zeros16
            return 0
        lax.fori_loop(0, NP // LANES, zside, 0)

        if mode == "l2":
            pltpu.sync_copy(invdeg, invdv)

        @pl.when(s < _ZT)
        def _():
            for b in range(ZROWS // _ZR):
                pltpu.async_copy(
                    zbuf, acc.at[pl.ds(s * ZROWS + b * _ZR, _ZR)], semz)
            for b in range(ZROWS // _ZR):
                pltpu.make_async_copy(
                    zbuf, acc.at[pl.ds(s * ZROWS + b * _ZR, _ZR)],
                    semz).wait()
        plsc.subcore_barrier()

        ones16 = jnp.ones((LANES,), jnp.float32)
        ebase = s * EP if mode == "l2" else c * EPC + s * EP

        def load_idx(q, b):
            off = ebase + q * _K
            pltpu.sync_copy(srcarr.at[pl.ds(off, _K)], srcv.at[b])
            pltpu.sync_copy(dst.at[pl.ds(off, _K)], dstv.at[b])

        def start_gather(b):
            if mode == "l2":
                @pl.when(c == 0)
                def _():
                    pltpu.async_copy(ha.at[srcv.at[b]], rows.at[b], semg[b])

                @pl.when(c == 1)
                def _():
                    pltpu.async_copy(hb.at[srcv.at[b]], rows.at[b], semg[b])
            else:
                pltpu.async_copy(ha.at[srcv.at[b]], rows.at[b], semg[b])

        def wait_gather(b):
            pltpu.make_async_copy(ha.at[srcv.at[b]], rows.at[b],
                                  semg[b]).wait()

        def consume(b):
            pltpu.sync_copy(rows.at[b], acc.at[dstv.at[b]], add=True)
            if mode == "l1":
                for j in range(_K // LANES):
                    d16 = dstv[b, pl.ds(j * LANES, LANES)]
                    plsc.addupdate_scatter(sidev, [d16], ones16)
            else:
                @pl.when(c == 1)
                def _():
                    for j in range(_K // LANES):
                        d16 = dstv[b, pl.ds(j * LANES, LANES)]
                        w16 = plsc.load_gather(invdv, [d16])
                        u16 = srcv[b, pl.ds(j * LANES, LANES)]
                        plsc.addupdate_scatter(sidev, [u16], w16)

        NPAIR = NCHUNK // 2
        load_idx(0, 0)
        start_gather(0)
        load_idx(1, 1)
        start_gather(1)

        def pair(p, _):
            for b in range(2):
                q = 2 * p + b
                wait_gather(b)
                consume(b)

                @pl.when(q + 2 < NCHUNK)
                def _():
                    load_idx(q + 2, b)
                    start_gather(b)
            return 0

        lax.fori_loop(0, NPAIR, pair, 0)
        if NCHUNK % 2 == 1:
            wait_gather(0)
            consume(0)

        pltpu.sync_copy(sidev, slab.at[c, s, 0])
        plsc.subcore_barrier()

        @pl.when(s < _ZT)
        def _():
            pltpu.sync_copy(acc.at[pl.ds(s * ZROWS, ZROWS)],
                            agg_out.at[c, pl.ds(s * ZROWS, ZROWS)])

        @pl.when(s < _RED_T)
        def _():
            base = s * _RED_N
            pltpu.sync_copy(slab.at[c, 0, 0, pl.ds(base, _RED_N)], red)
            for t in range(1, NS):
                pltpu.sync_copy(slab.at[c, t, 0, pl.ds(base, _RED_N)], tmp)

                def radd(j, _):
                    sl = pl.ds(j * LANES, LANES)
                    red[sl] = red[sl] + tmp[sl]
                    return 0
                lax.fori_loop(0, _RED_N // LANES, radd, 0)
            pltpu.sync_copy(red, side_out.at[c, 0, pl.ds(base, _RED_N)])

    NP = _RED_T * _RED_N
    out_types = (
        jax.ShapeDtypeStruct((2, N, W), jnp.float32),
        jax.ShapeDtypeStruct((2, 1, NP), jnp.float32),
    )
    return pl.kernel(body, out_type=out_types, mesh=mesh,
                     scratch_types=scratch,
                     compiler_params=pltpu.CompilerParams(
                         needs_layout_passes=False))


@functools.lru_cache(maxsize=None)
def _make_tc1(N, D_IN, D_HID, R):
    grid = (N // R,)
    H = D_HID // 2

    def body(x, agg, degt, Ws1, Wn1, b1, h1a_out, h1b_out, invd_out):
        deg = jnp.sum(degt[...], axis=1, keepdims=True)
        inv = 1.0 / jnp.clip(deg, 1.0, None)
        invd_out[...] = inv
        mean = (agg[0] + agg[1]) * inv
        h = (jnp.dot(x[...], Ws1[...], preferred_element_type=jnp.float32)
             + jnp.dot(mean, Wn1[...], preferred_element_type=jnp.float32)
             + b1[...])
        h = jnp.maximum(h, 0.0)
        nrm = jnp.sqrt(jnp.sum(h * h, axis=1, keepdims=True))
        h = h / jnp.maximum(nrm, 1e-12)
        h1a_out[...] = h[:, 0:H]
        h1b_out[...] = h[:, H:2 * H]

    return pl.pallas_call(
        body,
        grid=grid,
        in_specs=[
            pl.BlockSpec((R, D_IN), lambda i: (i, 0)),
            pl.BlockSpec((2, R, D_IN), lambda i: (0, i, 0)),
            pl.BlockSpec((R, 2), lambda i: (i, 0)),
            pl.BlockSpec((D_IN, D_HID), lambda i: (0, 0)),
            pl.BlockSpec((D_IN, D_HID), lambda i: (0, 0)),
            pl.BlockSpec((1, D_HID), lambda i: (0, 0)),
        ],
        out_specs=[
            pl.BlockSpec((R, H), lambda i: (i, 0)),
            pl.BlockSpec((R, H), lambda i: (i, 0)),
            pl.BlockSpec((R, 1), lambda i: (i, 0)),
        ],
        out_shape=[
            jax.ShapeDtypeStruct((N, H), jnp.float32),
            jax.ShapeDtypeStruct((N, H), jnp.float32),
            jax.ShapeDtypeStruct((N, 1), jnp.float32),
        ],
    )


@functools.lru_cache(maxsize=None)
def _make_tc2(N, D_HID, D_OUT, R):
    grid = (N // R,)
    H = D_HID // 2
    nblocks = N // R

    def body(h1a, h1b, agg, cvec, invd, Ws2, Wn2, b2, Ws3, Wn3, b3,
             out, s1, s2):
        i = pl.program_id(0)
        inv = invd[...]
        h = (jnp.dot(h1a[...], Ws2[0:H], preferred_element_type=jnp.float32)
             + jnp.dot(h1b[...], Ws2[H:2 * H],
                       preferred_element_type=jnp.float32)
             + jnp.dot(agg[0] * inv, Wn2[0:H],
                       preferred_element_type=jnp.float32)
             + jnp.dot(agg[1] * inv, Wn2[H:2 * H],
                       preferred_element_type=jnp.float32)
             + b2[...])
        h = jnp.maximum(h, 0.0)
        nrm = jnp.sqrt(jnp.sum(h * h, axis=1, keepdims=True))
        h2 = h / jnp.maximum(nrm, 1e-12)

        @pl.when(i == 0)
        def _():
            s1[...] = jnp.zeros_like(s1)
            s2[...] = jnp.zeros_like(s2)

        s1[...] += jnp.sum(h2, axis=0, keepdims=True)
        s2[...] += jnp.sum(cvec[...] * h2, axis=0, keepdims=True)

        @pl.when(i == nblocks - 1)
        def _():
            out[...] = (jnp.dot(s1[...], Ws3[...],
                                preferred_element_type=jnp.float32)
                        + jnp.dot(s2[...], Wn3[...],
                                  preferred_element_type=jnp.float32)
                        ) * (1.0 / N) + b3[...]

    return pl.pallas_call(
        body,
        grid=grid,
        in_specs=[
            pl.BlockSpec((R, H), lambda i: (i, 0)),
            pl.BlockSpec((R, H), lambda i: (i, 0)),
            pl.BlockSpec((2, R, H), lambda i: (0, i, 0)),
            pl.BlockSpec((R, 1), lambda i: (i, 0)),
            pl.BlockSpec((R, 1), lambda i: (i, 0)),
            pl.BlockSpec((D_HID, D_HID), lambda i: (0, 0)),
            pl.BlockSpec((D_HID, D_HID), lambda i: (0, 0)),
            pl.BlockSpec((1, D_HID), lambda i: (0, 0)),
            pl.BlockSpec((D_HID, D_OUT), lambda i: (0, 0)),
            pl.BlockSpec((D_HID, D_OUT), lambda i: (0, 0)),
            pl.BlockSpec((1, D_OUT), lambda i: (0, 0)),
        ],
        out_specs=pl.BlockSpec((1, D_OUT), lambda i: (0, 0)),
        out_shape=jax.ShapeDtypeStruct((1, D_OUT), jnp.float32),
        scratch_shapes=[
            pltpu.VMEM((1, D_HID), jnp.float32),
            pltpu.VMEM((1, D_HID), jnp.float32),
        ],
    )


def kernel(x, edge_index, Ws1, Wn1, b1, Ws2, Wn2, b2, Ws3, Wn3, b3):
    N, D_IN = x.shape
    E = edge_index.shape[1]
    D_HID = Ws2.shape[0]
    D_OUT = Ws3.shape[1]
    R = 1000

    src = edge_index[0]
    dst = edge_index[1]

    agg1, degp = _make_sc_agg(N, E, D_IN, "l1")(x, src, dst)
    degt = degp[:, 0, :N].T

    h1a, h1b, invd = _make_tc1(N, D_IN, D_HID, R)(
        x, agg1, degt, Ws1, Wn1, b1.reshape(1, D_HID))

    agg2, cside = _make_sc_agg(N, E, D_HID // 2, "l2")(
        h1a, h1b, src, dst, invd.reshape(N))
    cvec = cside[1, 0, :N].reshape(N, 1)

    out = _make_tc2(N, D_HID, D_OUT, R)(
        h1a, h1b, agg2, cvec, invd, Ws2, Wn2, b2.reshape(1, D_HID),
        Ws3, Wn3, b3.reshape(1, D_OUT))
    return out.reshape(D_OUT)
